# baseline (speedup 1.0000x reference)
import functools

import jax
import jax.numpy as jnp
from jax import lax
from jax.experimental import pallas as pl
from jax.experimental.pallas import tpu as pltpu
from jax.experimental.pallas import tpu_sc as plsc

_BATCH = 16384
_EMB = 64
_NC = 2
_NS = 16
_NW = _NC * _NS
_BPW = _BATCH // _NW
_GROUPS = _BPW // 16


def _rsqrt_newton(x):
    i = lax.bitcast_convert_type(x, jnp.int32)
    i = jnp.int32(0x5F3759DF) - lax.shift_right_arithmetic(i, jnp.int32(1))
    y = lax.bitcast_convert_type(i, jnp.float32)
    half, three_half = jnp.float32(0.5), jnp.float32(1.5)
    for _ in range(3):
        y = y * (three_half - half * x * y * y)
    return y


def _sc_body(hi_hbm, ri_hbm, ti_hbm, ent_hbm, rel_hbm, out_hbm,
             hi_v, ri_v, ti_v, h_v, r_v, t_v, o_v, nrm_v, sem0, sem1, sem2):
    wid = lax.axis_index("s") * _NC + lax.axis_index("c")
    base = wid * _BPW

    pltpu.sync_copy(hi_hbm.at[pl.ds(base, _BPW)], hi_v)
    pltpu.sync_copy(ri_hbm.at[pl.ds(base, _BPW)], ri_v)
    pltpu.sync_copy(ti_hbm.at[pl.ds(base, _BPW)], ti_v)

    cph = pltpu.async_copy(ent_hbm.at[hi_v], h_v, sem0)
    cpr = pltpu.async_copy(rel_hbm.at[ri_v], r_v, sem1)
    cpt = pltpu.async_copy(ent_hbm.at[ti_v], t_v, sem2)
    cph.wait()
    cpr.wait()
    cpt.wait()

    eps = jnp.float32(1e-24)
    zeros = jnp.zeros((16,), jnp.float32)
    ones_i = jnp.full((16,), 1, jnp.int32)
    c512 = jnp.full((16,), _BPW, jnp.int32)
    c1024 = jnp.full((16,), 2 * _BPW, jnp.int32)

    def _chunks(ref, i):
        return [ref[i, pl.ds(k * 16, 16)] for k in range(4)]

    def _ssq(c):
        return c[0] * c[0] + c[1] * c[1] + c[2] * c[2] + c[3] * c[3]

    def zero_nrm(b, carry):
        nrm_v[pl.ds(b * 16, 16)] = zeros
        return carry

    lax.fori_loop(0, _GROUPS * 3, zero_nrm, 0)

    def ssq_group(g, carry):
        jv = jnp.full((16,), g * 16, jnp.int32)
        for j in range(16):
            i = g * 16 + j
            plsc.addupdate_scatter(nrm_v, [jv], _ssq(_chunks(h_v, i)))
            plsc.addupdate_scatter(nrm_v, [jv + c512], _ssq(_chunks(r_v, i)))
            plsc.addupdate_scatter(nrm_v, [jv + c1024], _ssq(_chunks(t_v, i)))
            jv = jv + ones_i
        return carry

    lax.fori_loop(0, _GROUPS, ssq_group, 0)

    def newton16(b, carry):
        nrm_v[pl.ds(b * 16, 16)] = _rsqrt_newton(
            jnp.maximum(nrm_v[pl.ds(b * 16, 16)], eps))
        return carry

    lax.fori_loop(0, _GROUPS * 3, newton16, 0)

    def zero_out(b, carry):
        o_v[pl.ds(b * 16, 16)] = zeros
        return carry

    lax.fori_loop(0, _GROUPS, zero_out, 0)

    def dist_group(g, carry):
        jv = jnp.full((16,), g * 16, jnp.int32)
        for j in range(16):
            i = g * 16 + j
            hc = _chunks(h_v, i)
            rc = _chunks(r_v, i)
            tc = _chunks(t_v, i)
            ih = plsc.load_gather(nrm_v, [jv])
            ir = plsc.load_gather(nrm_v, [jv + c512])
            it = plsc.load_gather(nrm_v, [jv + c1024])
            s = jnp.abs(hc[0] * ih + rc[0] * ir - tc[0] * it)
            for k in range(1, 4):
                s = s + jnp.abs(hc[k] * ih + rc[k] * ir - tc[k] * it)
            plsc.addupdate_scatter(o_v, [jv], s)
            jv = jv + ones_i
        return carry

    lax.fori_loop(0, _GROUPS, dist_group, 0)
    pltpu.sync_copy(o_v, out_hbm.at[pl.ds(base, _BPW)])


@functools.partial(
    pl.kernel,
    out_type=jax.ShapeDtypeStruct((_BATCH,), jnp.float32),
    mesh=plsc.VectorSubcoreMesh(core_axis_name="c", subcore_axis_name="s"),
    compiler_params=pltpu.CompilerParams(
        needs_layout_passes=False, use_tc_tiling_on_sc=False),
    scratch_types=[
        pltpu.VMEM((_BPW,), jnp.int32),
        pltpu.VMEM((_BPW,), jnp.int32),
        pltpu.VMEM((_BPW,), jnp.int32),
        pltpu.VMEM((_BPW, _EMB), jnp.float32),
        pltpu.VMEM((_BPW, _EMB), jnp.float32),
        pltpu.VMEM((_BPW, _EMB), jnp.float32),
        pltpu.VMEM((_BPW,), jnp.float32),
        pltpu.VMEM((3 * _BPW,), jnp.float32),
        pltpu.SemaphoreType.DMA,
        pltpu.SemaphoreType.DMA,
        pltpu.SemaphoreType.DMA,
    ],
)
def _transe_sc(*refs):
    _sc_body(*refs)


def kernel(edge, entity_embedding, relation_embedding):
    h_idx = edge[:, 0]
    r_idx = edge[:, 1]
    t_idx = edge[:, 2]
    return _transe_sc(h_idx, r_idx, t_idx, entity_embedding, relation_embedding)
